# baseline (speedup 1.0000x reference)
import functools

import jax
import jax.numpy as jnp
from jax import lax
from jax.experimental import pallas as pl
from jax.experimental.pallas import tpu as pltpu
from jax.experimental.pallas import tpu_sc as plsc

_D = 64
_NWORDS = 430080
_NBAGS = 21504
_WORKERS = 32
_BPW = _NBAGS // _WORKERS
_ACC_ROWS = _BPW + 1
_C = 512
_CB = 128
_NS = 16
_LOFF = _BPW + 24


def _sload(ref, i):
    return ref[pl.ds(i, 16)][0]


def _search_last_le(loff, limit, lo0):
    lo, hi = lo0, jnp.int32(_BPW)
    for _ in range(10):
        mid = (lo + hi + 1) // 2
        take = _sload(loff, mid) <= limit
        lo = jnp.where(take, mid, lo)
        hi = jnp.where(take, hi, mid - 1)
    return lo


def _sc_bag_means(words_pad, offsets_pad, table, zeros_rows):
    mesh = plsc.VectorSubcoreMesh(core_axis_name="c", subcore_axis_name="s")

    @functools.partial(
        pl.kernel,
        out_type=jax.ShapeDtypeStruct((_NBAGS, _D), jnp.float32),
        mesh=mesh,
        scratch_types=[
            pltpu.VMEM((_LOFF,), jnp.int32),
            pltpu.VMEM((2 * _C // _CB, _CB), jnp.int32),
            pltpu.VMEM((2 * _C // _CB, _CB), jnp.int32),
            pltpu.VMEM((_C, _D), jnp.float32),
            pltpu.VMEM((_C, _D), jnp.float32),
            pltpu.VMEM((96, _D), jnp.float32),
            pltpu.VMEM_SHARED((_NS * _ACC_ROWS, _D), jnp.float32),
            pltpu.SemaphoreType.DMA((_C // _CB,)),
            pltpu.SemaphoreType.DMA((_C // _CB,)),
            pltpu.SemaphoreType.DMA,
            pltpu.SemaphoreType.DMA,
            pltpu.SemaphoreType.DMA,
        ],
        compiler_params=pltpu.CompilerParams(use_tc_tiling_on_sc=False),
    )
    def k(words_ref, offs_ref, table_ref, zrows_ref, out_ref,
          loff, widx, sidx, rowsA, rowsB, fbuf, acc,
          sem_gA, sem_gB, sem_sA, sem_sB, sem_w):
        c = lax.axis_index("c")
        s = lax.axis_index("s")
        wid = c * _NS + s
        bag0 = wid * _BPW
        abase = s * _ACC_ROWS

        pltpu.sync_copy(offs_ref.at[pl.ds(bag0, _LOFF)], loff)
        pltpu.sync_copy(zrows_ref, acc.at[pl.ds(abase, _ACC_ROWS)])

        w_start = _sload(loff, 0)
        w_end = _sload(loff, _BPW)
        cs0 = (w_start // _C) * _C
        n_chunks = (w_end - cs0 + _C - 1) // _C
        iota = lax.iota(jnp.int32, 16)
        trash_v = jnp.zeros((16,), jnp.int32) + (abase + _BPW)

        def _gather(roff, rows, sem_g):
            return [pltpu.async_copy(table_ref.at[widx.at[roff + j]],
                                     rows.at[pl.ds(j * _CB, _CB)],
                                     sem_g.at[j])
                    for j in range(_C // _CB)]

        def _paint(cs, roff):
            pos_last = cs + _C - 1
            for g in range(_C // 16):
                sidx[roff + g // (_CB // 16),
                     pl.ds((g % (_CB // 16)) * 16, 16)] = trash_v
            b_lo = _search_last_le(loff, cs, jnp.int32(0))
            b_hi = _search_last_le(loff, pos_last, b_lo)

            @pl.loop(b_lo, b_hi + 1)
            def _bag(b):
                s0 = jnp.maximum(_sload(loff, b) - cs, 0)
                e0 = jnp.minimum(_sload(loff, b + 1) - cs, _C)
                sv = jnp.zeros((16,), jnp.int32) + (abase + b)

                @pl.loop(s0 // 16, (e0 + 15) // 16)
                def _grp(g):
                    gp = g * 16 + iota
                    row = roff + g // (_CB // 16)
                    col = (g % (_CB // 16)) * 16
                    mask = jnp.logical_and(gp >= s0, gp < e0)
                    cur = sidx[row, pl.ds(col, 16)]
                    sidx[row, pl.ds(col, 16)] = jnp.where(mask, sv, cur)

        def _scatter(gcps, rows, roff, sem_s):
            scps = []
            for j in range(_C // _CB):
                gcps[j].wait()
                scps.append(pltpu.async_copy(rows.at[pl.ds(j * _CB, _CB)],
                                             acc.at[sidx.at[roff + j]],
                                             sem_s, add=True))
            return scps

        NB = _C // _CB

        @pl.loop(0, (n_chunks + 1) // 2)
        def _pair(ip):
            csA = cs0 + (2 * ip) * _C
            csB = csA + _C
            csa = pl.multiple_of(csA, _C)
            pltpu.async_copy(words_ref.at[pl.ds(csa // _CB, 2 * NB)],
                             widx, sem_w).wait()
            gA = _gather(0, rowsA, sem_gA)
            gB = _gather(NB, rowsB, sem_gB)
            _paint(csA, 0)
            sA = _scatter(gA, rowsA, 0, sem_sA)
            _paint(csB, NB)
            sB = _scatter(gB, rowsB, NB, sem_sB)
            for cp in sA + sB:
                cp.wait()

        def fin_t(t, _):
            pltpu.async_copy(acc.at[pl.ds(abase + t * 96, 96)], fbuf,
                             sem_w).wait()

            def fin_b(b, _):
                i = t * 96 + b
                ov = loff[pl.ds(i, 16)]
                cnt = ov[1] - ov[0]
                den = jnp.maximum(
                    (jnp.zeros((16,), jnp.int32) + cnt).astype(jnp.float32),
                    1.0)
                for kk in range(_D // 16):
                    fbuf[b, pl.ds(kk * 16, 16)] = (
                        fbuf[b, pl.ds(kk * 16, 16)] / den)
                return 0

            lax.fori_loop(0, 96, fin_b, 0)
            pltpu.async_copy(fbuf, out_ref.at[pl.ds(bag0 + t * 96, 96)],
                             sem_w).wait()
            return 0

        lax.fori_loop(0, _BPW // 96, fin_t, 0)

    return k(words_pad, offsets_pad, table, zeros_rows)


def _tc_loss(means):
    x = means.reshape(_NBAGS // 21, 21, _D)

    def body(x_ref, o_ref):
        xx = x_ref[...]
        src = xx[:, 0, :]
        tgt = xx[:, 1:, :]
        scores = jnp.sum(tgt * src[:, None, :], axis=-1)
        m = jnp.max(scores, axis=1)
        lse = jnp.log(jnp.sum(jnp.exp(scores - m[:, None]), axis=1)) + m
        o_ref[...] = jnp.mean(lse - scores[:, 0]).reshape(1, 1)

    out = pl.pallas_call(
        body, out_shape=jax.ShapeDtypeStruct((1, 1), jnp.float32))(x)
    return out[0, 0]


def kernel(words, offsets, emb_table):
    words = words.astype(jnp.int32)
    offsets = offsets.astype(jnp.int32)
    words_pad = jnp.concatenate(
        [words, jnp.zeros((2 * _C + _CB,), jnp.int32)]).reshape(-1, _CB)
    offsets_pad = jnp.concatenate(
        [offsets, jnp.full((24,), _NWORDS, jnp.int32)])
    zeros_rows = jnp.zeros((_ACC_ROWS, _D), jnp.float32)
    means = _sc_bag_means(words_pad, offsets_pad,
                          emb_table.astype(jnp.float32), zeros_rows)
    return _tc_loss(means)


# speedup vs baseline: 1.0116x; 1.0116x over previous
import functools

import jax
import jax.numpy as jnp
from jax import lax
from jax.experimental import pallas as pl
from jax.experimental.pallas import tpu as pltpu
from jax.experimental.pallas import tpu_sc as plsc

_D = 64
_NWORDS = 430080
_NBAGS = 21504
_WORKERS = 32
_BPW = _NBAGS // _WORKERS
_ACC_ROWS = _BPW + 1
_C = 512
_CB = 128
_NS = 16
_LOFF = _BPW + 24


def _sload(ref, i):
    return ref[pl.ds(i, 16)][0]


def _search_last_le(loff, limit, lo0):
    lo, hi = lo0, jnp.int32(_BPW)
    for _ in range(10):
        mid = (lo + hi + 1) // 2
        take = _sload(loff, mid) <= limit
        lo = jnp.where(take, mid, lo)
        hi = jnp.where(take, hi, mid - 1)
    return lo


def _sc_bag_means(words_pad, offsets_pad, table, zeros_rows):
    mesh = plsc.VectorSubcoreMesh(core_axis_name="c", subcore_axis_name="s")

    @functools.partial(
        pl.kernel,
        out_type=jax.ShapeDtypeStruct((_NBAGS, _D), jnp.float32),
        mesh=mesh,
        scratch_types=[
            pltpu.VMEM((_LOFF,), jnp.int32),
            pltpu.VMEM((_C // _CB, _CB), jnp.int32),
            pltpu.VMEM((_C // _CB, _CB), jnp.int32),
            pltpu.VMEM((_C // _CB, _CB), jnp.int32),
            pltpu.VMEM((_C // _CB, _CB), jnp.int32),
            pltpu.VMEM((_C, _D), jnp.float32),
            pltpu.VMEM((_C, _D), jnp.float32),
            pltpu.VMEM((96, _D), jnp.float32),
            pltpu.VMEM_SHARED((_NS * _ACC_ROWS, _D), jnp.float32),
            pltpu.SemaphoreType.DMA((_C // _CB,)),
            pltpu.SemaphoreType.DMA((_C // _CB,)),
            pltpu.SemaphoreType.DMA,
            pltpu.SemaphoreType.DMA,
            pltpu.SemaphoreType.DMA,
        ],
        compiler_params=pltpu.CompilerParams(use_tc_tiling_on_sc=False),
    )
    def k(words_ref, offs_ref, table_ref, zrows_ref, out_ref,
          loff, widxA, widxB, sidxA, sidxB, rowsA, rowsB, fbuf, acc,
          sem_gA, sem_gB, sem_sA, sem_sB, sem_w):
        c = lax.axis_index("c")
        s = lax.axis_index("s")
        wid = c * _NS + s
        bag0 = wid * _BPW
        abase = s * _ACC_ROWS

        pltpu.sync_copy(offs_ref.at[pl.ds(bag0, _LOFF)], loff)
        pltpu.sync_copy(zrows_ref, acc.at[pl.ds(abase, _ACC_ROWS)])

        w_start = _sload(loff, 0)
        w_end = _sload(loff, _BPW)
        cs0 = (w_start // _C) * _C
        n_chunks = (w_end - cs0 + _C - 1) // _C
        iota = lax.iota(jnp.int32, 16)
        trash_v = jnp.zeros((16,), jnp.int32) + (abase + _BPW)

        def _stage_and_gather(cs, widx, rows, sem_g):
            csa = pl.multiple_of(cs, _C)
            pltpu.async_copy(words_ref.at[pl.ds(csa // _CB, _C // _CB)],
                             widx, sem_w).wait()
            return [pltpu.async_copy(table_ref.at[widx.at[j]],
                                     rows.at[pl.ds(j * _CB, _CB)],
                                     sem_g.at[j])
                    for j in range(_C // _CB)]

        def _paint(cs, sidx):
            pos_last = cs + _C - 1
            for g in range(_C // 16):
                sidx[g // (_CB // 16), pl.ds((g % (_CB // 16)) * 16, 16)] = \
                    trash_v
            b_lo = _search_last_le(loff, cs, jnp.int32(0))
            b_hi = _search_last_le(loff, pos_last, b_lo)

            @pl.loop(b_lo, b_hi + 1)
            def _bag(b):
                s0 = jnp.maximum(_sload(loff, b) - cs, 0)
                e0 = jnp.minimum(_sload(loff, b + 1) - cs, _C)
                sv = jnp.zeros((16,), jnp.int32) + (abase + b)

                @pl.loop(s0 // 16, (e0 + 15) // 16)
                def _grp(g):
                    gp = g * 16 + iota
                    mask = jnp.logical_and(gp >= s0, gp < e0)
                    row = g // (_CB // 16)
                    col = (g % (_CB // 16)) * 16
                    cur = sidx[row, pl.ds(col, 16)]
                    sidx[row, pl.ds(col, 16)] = jnp.where(mask, sv, cur)

        def _scatter(gcps, rows, sidx, sem_s):
            scps = []
            for j in range(_C // _CB):
                gcps[j].wait()
                scps.append(pltpu.async_copy(rows.at[pl.ds(j * _CB, _CB)],
                                             acc.at[sidx.at[j]], sem_s,
                                             add=True))
            return scps

        @pl.loop(0, (n_chunks + 1) // 2)
        def _pair(ip):
            csA = cs0 + (2 * ip) * _C
            csB = csA + _C
            gA = _stage_and_gather(csA, widxA, rowsA, sem_gA)
            _paint(csA, sidxA)
            gB = _stage_and_gather(csB, widxB, rowsB, sem_gB)
            sA = _scatter(gA, rowsA, sidxA, sem_sA)
            _paint(csB, sidxB)
            sB = _scatter(gB, rowsB, sidxB, sem_sB)
            for cp in sA + sB:
                cp.wait()

        def fin_t(t, _):
            pltpu.async_copy(acc.at[pl.ds(abase + t * 96, 96)], fbuf,
                             sem_w).wait()

            def fin_b(b, _):
                i = t * 96 + b
                ov = loff[pl.ds(i, 16)]
                cnt = ov[1] - ov[0]
                den = jnp.maximum(
                    (jnp.zeros((16,), jnp.int32) + cnt).astype(jnp.float32),
                    1.0)
                for kk in range(_D // 16):
                    fbuf[b, pl.ds(kk * 16, 16)] = (
                        fbuf[b, pl.ds(kk * 16, 16)] / den)
                return 0

            lax.fori_loop(0, 96, fin_b, 0)
            pltpu.async_copy(fbuf, out_ref.at[pl.ds(bag0 + t * 96, 96)],
                             sem_w).wait()
            return 0

        lax.fori_loop(0, _BPW // 96, fin_t, 0)

    return k(words_pad, offsets_pad, table, zeros_rows)


def _tc_loss(means):
    x = means.reshape(_NBAGS // 21, 21, _D)

    def body(x_ref, o_ref):
        xx = x_ref[...]
        src = xx[:, 0, :]
        tgt = xx[:, 1:, :]
        scores = jnp.sum(tgt * src[:, None, :], axis=-1)
        m = jnp.max(scores, axis=1)
        lse = jnp.log(jnp.sum(jnp.exp(scores - m[:, None]), axis=1)) + m
        o_ref[...] = jnp.mean(lse - scores[:, 0]).reshape(1, 1)

    out = pl.pallas_call(
        body, out_shape=jax.ShapeDtypeStruct((1, 1), jnp.float32))(x)
    return out[0, 0]


def kernel(words, offsets, emb_table):
    words = words.astype(jnp.int32)
    offsets = offsets.astype(jnp.int32)
    words_pad = jnp.concatenate(
        [words, jnp.zeros((_C + _CB,), jnp.int32)]).reshape(-1, _CB)
    offsets_pad = jnp.concatenate(
        [offsets, jnp.full((24,), _NWORDS, jnp.int32)])
    zeros_rows = jnp.zeros((_ACC_ROWS, _D), jnp.float32)
    means = _sc_bag_means(words_pad, offsets_pad,
                          emb_table.astype(jnp.float32), zeros_rows)
    return _tc_loss(means)
